# baseline (speedup 1.0000x reference)
import functools

import jax
import jax.numpy as jnp
from jax import lax
from jax.experimental import pallas as pl
from jax.experimental.pallas import tpu as pltpu
from jax.experimental.pallas import tpu_sc as plsc

_N = 512
_E = 16384
_F = 32
_FLAT = _N * _F


def _sc_adj(src, dst):
    info = plsc.get_sparse_core_info()
    nc, ns, lanes = info.num_cores, info.num_subcores, info.num_lanes
    nw = nc * ns
    epw = _E // nw
    spw = (_N * _N) // ns
    mesh = plsc.VectorSubcoreMesh(core_axis_name="c", subcore_axis_name="s")

    @functools.partial(
        pl.kernel,
        mesh=mesh,
        out_type=jax.ShapeDtypeStruct((nc, _N * _N), jnp.float32),
        compiler_params=pltpu.CompilerParams(needs_layout_passes=False),
        scratch_types=[
            pltpu.VMEM((epw,), jnp.int32),
            pltpu.VMEM((epw,), jnp.int32),
            pltpu.VMEM((lanes,), jnp.float32),
            pltpu.VMEM_SHARED((_N * _N,), jnp.float32),
            pltpu.SemaphoreType.DMA,
        ],
    )
    def build(src_hbm, dst_hbm, z_hbm, out_hbm, src_v, dst_v, ones_v,
              acc_sh, sem):
        c = lax.axis_index("c")
        s = lax.axis_index("s")
        wid = c * ns + s
        pltpu.sync_copy(src_hbm.at[pl.ds(wid * epw, epw)], src_v)
        pltpu.sync_copy(dst_hbm.at[pl.ds(wid * epw, epw)], dst_v)
        ones_v[...] = jnp.ones((lanes,), jnp.float32)
        pltpu.sync_copy(z_hbm, acc_sh.at[pl.ds(s * spw, spw)])
        plsc.subcore_barrier()
        cps = []
        for j in range(epw // lanes):
            s16 = src_v[pl.ds(j * lanes, lanes)]
            d16 = dst_v[pl.ds(j * lanes, lanes)]
            idx = d16 * _N + s16
            cps.append(pltpu.async_copy(ones_v, acc_sh.at[idx], sem,
                                        add=True))
        for cp in cps:
            cp.wait()
        plsc.subcore_barrier()
        pltpu.sync_copy(acc_sh.at[pl.ds(s * spw, spw)],
                        out_hbm.at[c, pl.ds(s * spw, spw)])

    return build(src, dst, jnp.zeros((spw,), jnp.float32))


def _gcn_body(a_ref, x_ref, win_ref, bin_ref, wb_ref, bb_ref, h_ref):
    m = a_ref[0] + a_ref[1]
    row = lax.broadcasted_iota(jnp.int32, (_N, _N), 0)
    col = lax.broadcasted_iota(jnp.int32, (_N, _N), 1)
    m = m + jnp.where(row == col, 1.0, 0.0).astype(jnp.float32)
    deg = jnp.sum(m, axis=1, keepdims=True)
    dinv = lax.rsqrt(deg)

    def conv(h, w, b):
        hw = jnp.dot(h, w, preferred_element_type=jnp.float32)
        agg = jnp.dot(m, dinv * hw, preferred_element_type=jnp.float32)
        return dinv * agg + b

    h = jax.nn.relu(conv(x_ref[...], win_ref[...], bin_ref[...]))
    for i in range(3):
        t = jax.nn.relu(conv(h, wb_ref[2 * i], bb_ref[2 * i:2 * i + 1, :]))
        t = conv(t, wb_ref[2 * i + 1], bb_ref[2 * i + 1:2 * i + 2, :])
        h = jax.nn.relu(t + h)
    h_ref[...] = h


def _gcn(a, x, w_in, b_in, wb, bb):
    return pl.pallas_call(
        _gcn_body,
        out_shape=jax.ShapeDtypeStruct((_N, _F), jnp.float32),
    )(a, x, w_in, b_in, wb, bb)


_KB1 = 1024
_NB1 = 1024


def _mlp_body(v_ref, w1_ref, b1_ref, w2_ref, b2_ref, w3_ref, b3_ref, o_ref,
              acc1_ref, acc2_ref):
    nb = pl.program_id(0)
    kb = pl.program_id(1)

    @pl.when(kb == 0)
    def _():
        acc1_ref[...] = jnp.zeros_like(acc1_ref)

    acc1_ref[...] += jnp.dot(v_ref[...], w1_ref[...],
                             preferred_element_type=jnp.float32)

    @pl.when(kb == pl.num_programs(1) - 1)
    def _():
        @pl.when(nb == 0)
        def _():
            acc2_ref[...] = jnp.zeros_like(acc2_ref)

        v1 = jax.nn.relu(acc1_ref[...] + b1_ref[...])
        acc2_ref[...] += jnp.dot(v1, w2_ref[...],
                                 preferred_element_type=jnp.float32)

        @pl.when(nb == pl.num_programs(0) - 1)
        def _():
            u = jax.nn.relu(acc2_ref[...] + b2_ref[...])
            logits = jnp.dot(u, w3_ref[...],
                             preferred_element_type=jnp.float32) + b3_ref[...]
            mx = jnp.max(logits, axis=-1, keepdims=True)
            e = jnp.exp(logits - mx)
            o_ref[...] = e / jnp.sum(e, axis=-1, keepdims=True)


def _mlp(v, w1, b1, w2, b2, w3, b3):
    n_mid = w2.shape[1]
    n_out = w3.shape[1]
    return pl.pallas_call(
        _mlp_body,
        grid=(w1.shape[1] // _NB1, _FLAT // _KB1),
        in_specs=[
            pl.BlockSpec((1, _KB1), lambda nb, kb: (0, kb)),
            pl.BlockSpec((_KB1, _NB1), lambda nb, kb: (kb, nb)),
            pl.BlockSpec((1, _NB1), lambda nb, kb: (0, nb)),
            pl.BlockSpec((_NB1, n_mid), lambda nb, kb: (nb, 0)),
            pl.BlockSpec((1, n_mid), lambda nb, kb: (0, 0)),
            pl.BlockSpec((w3.shape[0], n_out), lambda nb, kb: (0, 0)),
            pl.BlockSpec((1, n_out), lambda nb, kb: (0, 0)),
        ],
        out_specs=pl.BlockSpec((1, n_out), lambda nb, kb: (0, 0)),
        out_shape=jax.ShapeDtypeStruct((1, n_out), jnp.float32),
        scratch_shapes=[
            pltpu.VMEM((1, _NB1), jnp.float32),
            pltpu.VMEM((1, n_mid), jnp.float32),
        ],
    )(v, w1, b1, w2, b2, w3, b3)


def kernel(x, edge_index, W_in, b_in, Wb, bb, W1, b1, W2, b2, W3, b3):
    a = _sc_adj(edge_index[0], edge_index[1]).reshape(2, _N, _N)
    h = _gcn(a, x, W_in, b_in.reshape(1, -1),
             Wb.reshape(6, _F, _F), bb.reshape(6, _F))
    v = h.reshape(1, _FLAT)
    out = _mlp(v, W1, b1.reshape(1, -1), W2, b2.reshape(1, -1),
               W3, b3.reshape(1, -1))
    return out.reshape(-1)


# speedup vs baseline: 1.0672x; 1.0672x over previous
import functools

import jax
import jax.numpy as jnp
from jax import lax
from jax.experimental import pallas as pl
from jax.experimental.pallas import tpu as pltpu
from jax.experimental.pallas import tpu_sc as plsc

_N = 512
_E = 16384
_F = 32
_FLAT = _N * _F


def _sc_adj(src, dst):
    info = plsc.get_sparse_core_info()
    nc, ns, lanes = info.num_cores, info.num_subcores, info.num_lanes
    nw = nc * ns
    epw = _E // nw
    spw = (_N * _N) // ns
    mesh = plsc.VectorSubcoreMesh(core_axis_name="c", subcore_axis_name="s")

    @functools.partial(
        pl.kernel,
        mesh=mesh,
        out_type=jax.ShapeDtypeStruct((nc, _N * _N), jnp.float32),
        compiler_params=pltpu.CompilerParams(needs_layout_passes=False),
        scratch_types=[
            pltpu.VMEM((epw,), jnp.int32),
            pltpu.VMEM((epw,), jnp.int32),
            pltpu.VMEM((lanes,), jnp.float32),
            pltpu.VMEM_SHARED((_N * _N,), jnp.float32),
            pltpu.SemaphoreType.DMA,
        ],
    )
    def build(src_hbm, dst_hbm, z_hbm, out_hbm, src_v, dst_v, ones_v,
              acc_sh, sem):
        c = lax.axis_index("c")
        s = lax.axis_index("s")
        wid = c * ns + s
        pltpu.sync_copy(src_hbm.at[pl.ds(wid * epw, epw)], src_v)
        pltpu.sync_copy(dst_hbm.at[pl.ds(wid * epw, epw)], dst_v)
        ones_v[...] = jnp.ones((lanes,), jnp.float32)
        pltpu.sync_copy(z_hbm, acc_sh.at[pl.ds(s * spw, spw)])
        plsc.subcore_barrier()
        cps = []
        for j in range(epw // lanes):
            s16 = src_v[pl.ds(j * lanes, lanes)]
            d16 = dst_v[pl.ds(j * lanes, lanes)]
            idx = d16 * _N + s16
            cps.append(pltpu.async_copy(ones_v, acc_sh.at[idx], sem,
                                        add=True))
        for cp in cps:
            cp.wait()
        plsc.subcore_barrier()
        pltpu.sync_copy(acc_sh.at[pl.ds(s * spw, spw)],
                        out_hbm.at[c, pl.ds(s * spw, spw)])

    return build(src, dst, jnp.zeros((spw,), jnp.float32))


def _gcn_body(a_ref, x_ref, win_ref, bin_ref, wb_ref, bb_ref, h_ref):
    m = a_ref[0] + a_ref[1]
    row = lax.broadcasted_iota(jnp.int32, (_N, _N), 0)
    col = lax.broadcasted_iota(jnp.int32, (_N, _N), 1)
    m = m + jnp.where(row == col, 1.0, 0.0).astype(jnp.float32)
    deg = jnp.sum(m, axis=1, keepdims=True)
    dinv = lax.rsqrt(deg)

    def conv(h, w, b):
        hw = jnp.dot(h, w, preferred_element_type=jnp.float32)
        agg = jnp.dot(m, dinv * hw, preferred_element_type=jnp.float32)
        return dinv * agg + b

    h = jax.nn.relu(conv(x_ref[...], win_ref[...], bin_ref[...]))
    for i in range(3):
        t = jax.nn.relu(conv(h, wb_ref[2 * i], bb_ref[2 * i:2 * i + 1, :]))
        t = conv(t, wb_ref[2 * i + 1], bb_ref[2 * i + 1:2 * i + 2, :])
        h = jax.nn.relu(t + h)
    h_ref[...] = h


def _gcn(a, x, w_in, b_in, wb, bb):
    return pl.pallas_call(
        _gcn_body,
        out_shape=jax.ShapeDtypeStruct((_N, _F), jnp.float32),
    )(a, x, w_in, b_in, wb, bb)


_KB1 = 2048
_NB1 = 1024


def _mlp_body(v_ref, w1_ref, b1_ref, w2_ref, b2_ref, w3_ref, b3_ref, o_ref,
              acc1_ref, acc2_ref):
    nb = pl.program_id(0)
    kb = pl.program_id(1)

    @pl.when(kb == 0)
    def _():
        acc1_ref[...] = jnp.zeros_like(acc1_ref)

    acc1_ref[...] += jnp.dot(v_ref[...], w1_ref[...],
                             preferred_element_type=jnp.float32)

    @pl.when(kb == pl.num_programs(1) - 1)
    def _():
        @pl.when(nb == 0)
        def _():
            acc2_ref[...] = jnp.zeros_like(acc2_ref)

        v1 = jax.nn.relu(acc1_ref[...] + b1_ref[...])
        acc2_ref[...] += jnp.dot(v1, w2_ref[...],
                                 preferred_element_type=jnp.float32)

        @pl.when(nb == pl.num_programs(0) - 1)
        def _():
            u = jax.nn.relu(acc2_ref[...] + b2_ref[...])
            logits = jnp.dot(u, w3_ref[...],
                             preferred_element_type=jnp.float32) + b3_ref[...]
            mx = jnp.max(logits, axis=-1, keepdims=True)
            e = jnp.exp(logits - mx)
            o_ref[...] = e / jnp.sum(e, axis=-1, keepdims=True)


def _mlp(v, w1, b1, w2, b2, w3, b3):
    n_mid = w2.shape[1]
    n_out = w3.shape[1]
    return pl.pallas_call(
        _mlp_body,
        grid=(w1.shape[1] // _NB1, _FLAT // _KB1),
        in_specs=[
            pl.BlockSpec((1, _KB1), lambda nb, kb: (0, kb)),
            pl.BlockSpec((_KB1, _NB1), lambda nb, kb: (kb, nb)),
            pl.BlockSpec((1, _NB1), lambda nb, kb: (0, nb)),
            pl.BlockSpec((_NB1, n_mid), lambda nb, kb: (nb, 0)),
            pl.BlockSpec((1, n_mid), lambda nb, kb: (0, 0)),
            pl.BlockSpec((w3.shape[0], n_out), lambda nb, kb: (0, 0)),
            pl.BlockSpec((1, n_out), lambda nb, kb: (0, 0)),
        ],
        out_specs=pl.BlockSpec((1, n_out), lambda nb, kb: (0, 0)),
        out_shape=jax.ShapeDtypeStruct((1, n_out), jnp.float32),
        scratch_shapes=[
            pltpu.VMEM((1, _NB1), jnp.float32),
            pltpu.VMEM((1, n_mid), jnp.float32),
        ],
    )(v, w1, b1, w2, b2, w3, b3)


def kernel(x, edge_index, W_in, b_in, Wb, bb, W1, b1, W2, b2, W3, b3):
    a = _sc_adj(edge_index[0], edge_index[1]).reshape(2, _N, _N)
    h = _gcn(a, x, W_in, b_in.reshape(1, -1),
             Wb.reshape(6, _F, _F), bb.reshape(6, _F))
    v = h.reshape(1, _FLAT)
    out = _mlp(v, W1, b1.reshape(1, -1), W2, b2.reshape(1, -1),
               W3, b3.reshape(1, -1))
    return out.reshape(-1)
